# Initial kernel scaffold; baseline (speedup 1.0000x reference)
import jax
import jax.numpy as jnp
from jax.experimental import pallas as pl


def kernel(x, ei_seq, ei_str_knn, ei_str_dis, ei_surf, ei_lrr, W1, b1, ln_w, ln_b, W2, b2):
    raise NotImplementedError("write your pallas kernel here")


# speedup vs baseline: 3.2823x; 3.2823x over previous
import functools

import jax
import jax.numpy as jnp
from jax import lax
from jax.experimental import pallas as pl
from jax.experimental.pallas import tpu as pltpu
from jax.experimental.pallas import tpu_sc as plsc

N = 10000
H = 128
E = 320000
ETYPES = 5
_ATT_BIAS = (-4.0, -4.0, -4.0, -4.0, -2.772)

NC, NS, L = 2, 16, 16
NW = NC * NS
K = 128
CPT = 79
EPT = CPT * K
EPAD = NW * EPT
HC = H + 16
NPAD = 10240
RPT = NPAD // NS
RQ = RPT // 4
DUMMY = N


def _sc_agg(xa, srcs, dsts):
    mesh = plsc.VectorSubcoreMesh(core_axis_name="c", subcore_axis_name="s")

    @functools.partial(
        pl.kernel,
        out_type=jax.ShapeDtypeStruct((NC, ETYPES, NPAD, HC), jnp.float32),
        mesh=mesh,
        scratch_types=[
            pltpu.VMEM((K,), jnp.int32),
            pltpu.VMEM((K,), jnp.int32),
            pltpu.VMEM((K, HC), jnp.float32),
            pltpu.VMEM_SHARED((NPAD, HC), jnp.float32),
            pltpu.SemaphoreType.DMA,
        ],
        compiler_params=pltpu.CompilerParams(use_tc_tiling_on_sc=False),
    )
    def k(xa_hbm, src_hbm, dst_hbm, out_hbm, src_v, dst_v, rows_v,
          sums_sh, sem):
        c = lax.axis_index("c")
        s = lax.axis_index("s")
        wid = c * NS + s
        zvec = jnp.zeros((L,), jnp.float32)

        def zrow(i, carry):
            for j in range(HC // L):
                rows_v[i, pl.ds(j * L, L)] = zvec
            return carry

        def zero_own_slice():
            lax.fori_loop(0, K, zrow, 0)
            for q in range(RPT // K):
                pltpu.sync_copy(rows_v, sums_sh.at[pl.ds(s * RPT + q * K, K)])

        zero_own_slice()

        for e in range(ETYPES):
            plsc.subcore_barrier()

            def chunk(i, carry):
                pltpu.sync_copy(src_hbm.at[e, wid, i], src_v)
                pltpu.sync_copy(dst_hbm.at[e, wid, i], dst_v)
                pltpu.async_copy(xa_hbm.at[src_v], rows_v, sem).wait()
                pltpu.sync_copy(rows_v, sums_sh.at[dst_v], add=True)
                return carry

            lax.fori_loop(0, CPT, chunk, 0)
            plsc.subcore_barrier()
            pltpu.sync_copy(sums_sh.at[pl.ds(s * RPT, RPT)],
                            out_hbm.at[c, e, pl.ds(s * RPT, RPT)])
            if e < ETYPES - 1:
                zero_own_slice()

    return k(xa, srcs, dsts)


_BN = 400
_GRID = N // _BN


def _attn_weights_kernel(blk_ref, w1_ref, b1_ref, lnw_ref, lnb_ref, w2_ref,
                         b2_ref, w_ref, acc_ref):
    i = pl.program_id(0)

    @pl.when(i == 0)
    def _():
        acc_ref[...] = jnp.zeros_like(acc_ref)

    blk = blk_ref[...]
    tot = blk[0] + blk[1]
    sums = tot[:, :, :H]
    cnt = jnp.maximum(tot[:, :, H], 1.0)
    agg = sums / cnt[:, :, None]
    acc_ref[...] += agg.sum(axis=1)

    @pl.when(i == pl.num_programs(0) - 1)
    def _():
        g = acc_ref[...] * (1.0 / N)
        h = g @ w1_ref[...] + b1_ref[...]
        mu = jnp.mean(h, axis=-1, keepdims=True)
        var = jnp.mean((h - mu) ** 2, axis=-1, keepdims=True)
        h = (h - mu) * lax.rsqrt(var + 1e-5) * lnw_ref[...] + lnb_ref[...]
        h = jnp.maximum(h, 0.0)
        scores = h @ w2_ref[...] + b2_ref[...]
        eidx = lax.broadcasted_iota(jnp.int32, (ETYPES, 1), 0)
        scores = scores + jnp.where(eidx == ETYPES - 1, _ATT_BIAS[-1],
                                    _ATT_BIAS[0])
        w = jax.nn.sigmoid(scores * 0.5) * 2.0
        w_ref[...] = jnp.clip(w, 0.05, 2.0)


def _combine_kernel(blk_ref, w_ref, out_ref):
    blk = blk_ref[...]
    tot = blk[0] + blk[1]
    sums = tot[:, :, :H]
    cnt = jnp.maximum(tot[:, :, H], 1.0)
    agg = sums / cnt[:, :, None]
    w = w_ref[...]
    out_ref[...] = jnp.sum(agg * w[:, :, None], axis=0)


def _tc_finish(psums, W1, b1, ln_w, ln_b, W2, b2):
    blk_spec = pl.BlockSpec((NC, ETYPES, _BN, HC), lambda i: (0, 0, i, 0))
    full = lambda shape: pl.BlockSpec(shape, lambda i: (0,) * len(shape))
    w = pl.pallas_call(
        _attn_weights_kernel,
        grid=(_GRID,),
        in_specs=[blk_spec, full((H, H // 4)), full((1, H // 4)),
                  full((1, H // 4)), full((1, H // 4)), full((H // 4, 1)),
                  full((1, 1))],
        out_specs=full((ETYPES, 1)),
        out_shape=jax.ShapeDtypeStruct((ETYPES, 1), jnp.float32),
        scratch_shapes=[pltpu.VMEM((ETYPES, H), jnp.float32)],
    )(psums, W1, b1.reshape(1, -1), ln_w.reshape(1, -1), ln_b.reshape(1, -1),
      W2, b2.reshape(1, -1))
    fused = pl.pallas_call(
        _combine_kernel,
        grid=(_GRID,),
        in_specs=[blk_spec, full((ETYPES, 1))],
        out_specs=pl.BlockSpec((_BN, H), lambda i: (i, 0)),
        out_shape=jax.ShapeDtypeStruct((N, H), jnp.float32),
    )(psums, w)
    return fused


def kernel(x, ei_seq, ei_str_knn, ei_str_dis, ei_surf, ei_lrr,
           W1, b1, ln_w, ln_b, W2, b2):
    xa = jnp.concatenate(
        [x, jnp.ones((N, 1), jnp.float32), jnp.zeros((N, HC - H - 1),
                                                     jnp.float32)], axis=1)
    srcs, dsts = [], []
    pad = EPAD - E
    for ei in (ei_seq, ei_str_knn, ei_str_dis, ei_surf, ei_lrr):
        srcs.append(jnp.concatenate(
            [ei[0], jnp.zeros((pad,), jnp.int32)]).reshape(NW, CPT, K))
        dsts.append(jnp.concatenate(
            [ei[1], jnp.full((pad,), DUMMY, jnp.int32)]).reshape(NW, CPT, K))
    psums = _sc_agg(xa, jnp.stack(srcs), jnp.stack(dsts))
    return _tc_finish(psums, W1, b1, ln_w, ln_b, W2, b2)
